# Initial kernel scaffold; baseline (speedup 1.0000x reference)
import jax
import jax.numpy as jnp
from jax.experimental import pallas as pl


def kernel(support_set, support_emb, edge_index, edge_type, node_emb, W_rel, W_self, W1, b1, W2, b2):
    raise NotImplementedError("write your pallas kernel here")


# speedup vs baseline: 36.4204x; 36.4204x over previous
import functools

import jax
import jax.numpy as jnp
from jax import lax
from jax.experimental import pallas as pl
from jax.experimental.pallas import tpu as pltpu
from jax.experimental.pallas import tpu_sc as plsc

N = 10000
E = 320000
D = 128
R = 8
B = 64
K = 5

NPOS = B * K * 2
PPB = K * 2
TRASH = NPOS
S_PAD = 672
ACC_ROWS = S_PAD * R
N_PAD = 10240
NW = 32
EPT = E // NW
ECHUNK = 2000
CHUNK = 128
SEL_CAP = N_PAD
SPT = 24
SPT_TILES = S_PAD // SPT
ACC_RPT = ACC_ROWS // 16
DEG_ROWS = 768
DEG_RPT = DEG_ROWS // 16


def _sc_kernel(dst_h, src_h, typ_h, slot_h, emb_h, idx_h, zrows_h, ones_h,
               acc_o, deg_o, posl_o, embs_o,
               slot_v, dst_v, src_v, typ_v, sela_v, sels_v,
               rows_v, achk_v, schk_v, ones_v, idx_v, posl_v, rows24_v,
               acc_sh, deg_sh):
    c = lax.axis_index("c")
    s = lax.axis_index("s")
    wid = s * 2 + c
    i16 = lax.iota(jnp.int32, 16)
    zero16i = i16 * 0
    trash16a = zero16i + TRASH * R

    pltpu.sync_copy(slot_h, slot_v)
    pltpu.sync_copy(idx_h, idx_v)

    pltpu.sync_copy(zrows_h, acc_sh.at[pl.ds(s * ACC_RPT, ACC_RPT)])
    pltpu.sync_copy(zrows_h.at[pl.ds(0, DEG_RPT)],
                    deg_sh.at[pl.ds(s * DEG_RPT, DEG_RPT)])
    pltpu.sync_copy(ones_h, ones_v)

    plsc.subcore_barrier()

    def comp_chunk(ec, off):
        base = wid * EPT + ec * ECHUNK
        pltpu.sync_copy(dst_h.at[pl.ds(base, ECHUNK)], dst_v)
        pltpu.sync_copy(src_h.at[pl.ds(base, ECHUNK)], src_v)
        pltpu.sync_copy(typ_h.at[pl.ds(base, ECHUNK)], typ_v)

        def comp_body(i, off):
            d = dst_v[pl.ds(i * 16, 16)]
            sl = plsc.load_gather(slot_v, [d])
            r = typ_v[pl.ds(i * 16, 16)]
            sv = src_v[pl.ds(i * 16, 16)]
            a = sl * R + r
            msk = sl < TRASH
            mi = msk.astype(jnp.int32)
            pos = off + jnp.cumsum(mi) - mi
            plsc.store_scatter(sels_v, [pos], sv, mask=msk)
            plsc.store_scatter(sela_v, [pos], a, mask=msk)
            return off + jnp.sum(mi)

        return lax.fori_loop(0, ECHUNK // 16, comp_body, off)

    n_sel = lax.fori_loop(0, EPT // ECHUNK, comp_chunk, jnp.int32(0))

    for j in range(CHUNK // 16 + 1):
        pos = n_sel + j * 16 + i16
        mt = pos < SEL_CAP
        plsc.store_scatter(sels_v, [pos], zero16i, mask=mt)
        plsc.store_scatter(sela_v, [pos], trash16a, mask=mt)

    n_chunks = lax.shift_right_logical(n_sel + (CHUNK - 1), 7)

    def chunk_body(g, carry):
        for j in range(CHUNK // 16):
            ach = sela_v[pl.ds(g * CHUNK + j * 16, 16)]
            achk_v[pl.ds(j * 16, 16)] = ach
            schk_v[pl.ds(j * 16, 16)] = lax.shift_right_logical(ach, 3)
        pltpu.sync_copy(emb_h.at[sels_v.at[pl.ds(g * CHUNK, CHUNK)]], rows_v)
        pltpu.sync_copy(rows_v, acc_sh.at[achk_v], add=True)
        pltpu.sync_copy(ones_v, deg_sh.at[schk_v], add=True)
        return carry

    lax.fori_loop(0, n_chunks, chunk_body, jnp.int32(0))

    @pl.when(wid == 0)
    def _():
        for j in range(B):
            posl_v[j, pl.ds(0, 16)] = zero16i + TRASH
        for g in range(NPOS // 16):
            sl = plsc.load_gather(slot_v, [idx_v[pl.ds(g * 16, 16)]])
            p = i16 + g * 16
            rvec = lax.shift_right_logical(p * 6554, 16)
            cvec = p - rvec * PPB
            plsc.store_scatter(posl_v, [rvec, cvec], sl)
        pltpu.sync_copy(posl_v, posl_o)

    @pl.when(wid < SPT_TILES)
    def _():
        pltpu.sync_copy(emb_h.at[idx_v.at[pl.ds(wid * SPT, SPT)]], rows24_v)

    plsc.subcore_barrier()

    pltpu.sync_copy(acc_sh.at[pl.ds(s * ACC_RPT, ACC_RPT)],
                    acc_o.at[c, pl.ds(s * ACC_RPT, ACC_RPT)])
    pltpu.sync_copy(deg_sh.at[pl.ds(s * DEG_RPT, DEG_RPT)],
                    deg_o.at[c, pl.ds(s * DEG_RPT, DEG_RPT)])

    @pl.when(wid < SPT_TILES)
    def _():
        pltpu.sync_copy(rows24_v, embs_o.at[pl.ds(wid * SPT, SPT)])


_sc_call = functools.partial(
    pl.kernel,
    mesh=plsc.VectorSubcoreMesh(core_axis_name="c", subcore_axis_name="s"),
    compiler_params=pltpu.CompilerParams(needs_layout_passes=False),
    out_type=(
        jax.ShapeDtypeStruct((2, ACC_ROWS, D), jnp.float32),
        jax.ShapeDtypeStruct((2, DEG_ROWS, D), jnp.float32),
        jax.ShapeDtypeStruct((B, 16), jnp.int32),
        jax.ShapeDtypeStruct((S_PAD, D), jnp.float32),
    ),
    scratch_types=[
        pltpu.VMEM((N_PAD,), jnp.int32),
        pltpu.VMEM((ECHUNK,), jnp.int32),
        pltpu.VMEM((ECHUNK,), jnp.int32),
        pltpu.VMEM((ECHUNK,), jnp.int32),
        pltpu.VMEM((SEL_CAP,), jnp.int32),
        pltpu.VMEM((SEL_CAP,), jnp.int32),
        pltpu.VMEM((CHUNK, D), jnp.float32),
        pltpu.VMEM((CHUNK,), jnp.int32),
        pltpu.VMEM((CHUNK,), jnp.int32),
        pltpu.VMEM((CHUNK, D), jnp.float32),
        pltpu.VMEM((S_PAD + 32,), jnp.int32),
        pltpu.VMEM((B, 16), jnp.int32),
        pltpu.VMEM((SPT, D), jnp.float32),
        pltpu.VMEM_SHARED((ACC_ROWS, D), jnp.float32),
        pltpu.VMEM_SHARED((DEG_ROWS, D), jnp.float32),
    ],
)(_sc_kernel)


def _tc_kernel(acc_ref, deg_ref, posl_ref, embs_ref,
               wrel_ref, wself_ref, w1_ref, b1_ref, w2_ref, b2_ref, out_ref):
    acc = acc_ref[0] + acc_ref[1]
    agg = jnp.dot(acc, wrel_ref[...], preferred_element_type=jnp.float32)
    deg = deg_ref[0, 0:S_PAD, 0:1] + deg_ref[1, 0:S_PAD, 0:1]
    agg = agg / jnp.maximum(deg, 1.0)
    h = jax.nn.relu(agg + jnp.dot(embs_ref[...], wself_ref[...],
                                  preferred_element_type=jnp.float32))
    ps = posl_ref[...]
    iot = lax.broadcasted_iota(jnp.int32, (B, S_PAD), 1)
    pool = jnp.zeros((B, S_PAD), jnp.float32)
    for j in range(PPB):
        pool += (ps[:, j:j + 1] == iot).astype(jnp.float32)
    mean = jnp.dot(pool, h, preferred_element_type=jnp.float32) * (1.0 / PPB)
    hidden = jax.nn.relu(jnp.dot(mean, w1_ref[...],
                                 preferred_element_type=jnp.float32)
                         + b1_ref[...])
    out_ref[...] = jax.nn.sigmoid(
        jnp.dot(hidden, w2_ref[...], preferred_element_type=jnp.float32)
        + b2_ref[...])


def kernel(support_set, support_emb, edge_index, edge_type, node_emb,
           W_rel, W_self, W1, b1, W2, b2):
    del support_emb
    src = edge_index[0].astype(jnp.int32)
    dst = edge_index[1].astype(jnp.int32)
    typ = edge_type.astype(jnp.int32)

    idx = support_set[:, :, jnp.array([0, 2])].astype(jnp.int32).reshape(-1)
    idx_pad = jnp.concatenate(
        [idx, jnp.zeros((S_PAD + 32 - NPOS,), jnp.int32)])
    slot_map = jnp.full((N_PAD,), TRASH, jnp.int32).at[idx].set(
        jnp.arange(NPOS, dtype=jnp.int32))
    zrows = jnp.zeros((ACC_RPT, D), jnp.float32)
    ones = jnp.ones((CHUNK, D), jnp.float32)

    acc, deg, posl, emb_slots = _sc_call(
        dst, src, typ, slot_map, node_emb, idx_pad, zrows, ones)

    acc2 = acc.reshape(2, S_PAD, R * D)
    wrel2 = W_rel.reshape(R * D, D)

    out = pl.pallas_call(
        _tc_kernel,
        out_shape=jax.ShapeDtypeStruct((B, 1), jnp.float32),
    )(acc2, deg, posl, emb_slots, wrel2, W_self, W1,
      b1.reshape(1, D), W2, b2.reshape(1, 1))
    return out


# speedup vs baseline: 38.0037x; 1.0435x over previous
import functools

import jax
import jax.numpy as jnp
from jax import lax
from jax.experimental import pallas as pl
from jax.experimental.pallas import tpu as pltpu
from jax.experimental.pallas import tpu_sc as plsc

N = 10000
E = 320000
D = 128
R = 8
B = 64
K = 5

NPOS = B * K * 2
PPB = K * 2
TRASH = NPOS
S_PAD = 672
ACC_ROWS = S_PAD * R
N_PAD = 10240
NW = 32
EPT = E // NW
ECHUNK = 2000
CHUNK = 128
SEL_CAP = N_PAD
SPT = 24
SPT_TILES = S_PAD // SPT
ACC_RPT = ACC_ROWS // 16
DEG_ROWS = 768
DEG_RPT = DEG_ROWS // 16


def _sc_kernel(dst_h, src_h, typ_h, slot_h, emb_h, idx_h, zrows_h,
               acc_o, deg_o, posl_o, embs_o,
               slot_v, dst_v, src_v, typ_v, sela_v, sels_v,
               rows_v, achk_v, cnt_v, idx_v, posl_v, rows24_v,
               acc_sh):
    c = lax.axis_index("c")
    s = lax.axis_index("s")
    wid = s * 2 + c
    i16 = lax.iota(jnp.int32, 16)
    zero16i = i16 * 0
    trash16a = zero16i + TRASH * R

    pltpu.sync_copy(slot_h, slot_v)
    pltpu.sync_copy(idx_h, idx_v)

    pltpu.sync_copy(zrows_h, acc_sh.at[pl.ds(s * ACC_RPT, ACC_RPT)])
    zero16f = (i16 * 0).astype(jnp.float32)
    for j in range(DEG_ROWS // 16):
        cnt_v[pl.ds(j * 16, 16)] = zero16f

    plsc.subcore_barrier()

    def comp_chunk(ec, off):
        base = wid * EPT + ec * ECHUNK
        pltpu.sync_copy(dst_h.at[pl.ds(base, ECHUNK)], dst_v)
        pltpu.sync_copy(src_h.at[pl.ds(base, ECHUNK)], src_v)
        pltpu.sync_copy(typ_h.at[pl.ds(base, ECHUNK)], typ_v)

        def comp_body(i, off):
            d = dst_v[pl.ds(i * 16, 16)]
            sl = plsc.load_gather(slot_v, [d])
            r = typ_v[pl.ds(i * 16, 16)]
            sv = src_v[pl.ds(i * 16, 16)]
            a = sl * R + r
            msk = sl < TRASH
            mi = msk.astype(jnp.int32)
            pos = off + jnp.cumsum(mi) - mi
            plsc.store_scatter(sels_v, [pos], sv, mask=msk)
            plsc.store_scatter(sela_v, [pos], a, mask=msk)
            counts, lastm = plsc.scan_count(sl, mask=msk)
            plsc.addupdate_scatter(cnt_v, [sl], counts.astype(jnp.float32),
                                   mask=lastm)
            return off + jnp.sum(mi)

        return lax.fori_loop(0, ECHUNK // 16, comp_body, off)

    n_sel = lax.fori_loop(0, EPT // ECHUNK, comp_chunk, jnp.int32(0))

    for j in range(CHUNK // 16 + 1):
        pos = n_sel + j * 16 + i16
        mt = pos < SEL_CAP
        plsc.store_scatter(sels_v, [pos], zero16i, mask=mt)
        plsc.store_scatter(sela_v, [pos], trash16a, mask=mt)

    n_chunks = lax.shift_right_logical(n_sel + (CHUNK - 1), 7)

    def chunk_body(g, carry):
        for j in range(CHUNK // 16):
            achk_v[pl.ds(j * 16, 16)] = sela_v[pl.ds(g * CHUNK + j * 16, 16)]
        pltpu.sync_copy(emb_h.at[sels_v.at[pl.ds(g * CHUNK, CHUNK)]], rows_v)
        pltpu.sync_copy(rows_v, acc_sh.at[achk_v], add=True)
        return carry

    lax.fori_loop(0, n_chunks, chunk_body, jnp.int32(0))

    @pl.when(wid == 0)
    def _():
        for j in range(B):
            posl_v[j, pl.ds(0, 16)] = zero16i + TRASH
        for g in range(NPOS // 16):
            sl = plsc.load_gather(slot_v, [idx_v[pl.ds(g * 16, 16)]])
            p = i16 + g * 16
            rvec = lax.shift_right_logical(p * 6554, 16)
            cvec = p - rvec * PPB
            plsc.store_scatter(posl_v, [rvec, cvec], sl)
        pltpu.sync_copy(posl_v, posl_o)

    @pl.when(wid < SPT_TILES)
    def _():
        pltpu.sync_copy(emb_h.at[idx_v.at[pl.ds(wid * SPT, SPT)]], rows24_v)

    plsc.subcore_barrier()

    pltpu.sync_copy(acc_sh.at[pl.ds(s * ACC_RPT, ACC_RPT)],
                    acc_o.at[c, pl.ds(s * ACC_RPT, ACC_RPT)])
    pltpu.sync_copy(cnt_v, deg_o.at[wid])

    @pl.when(wid < SPT_TILES)
    def _():
        pltpu.sync_copy(rows24_v, embs_o.at[pl.ds(wid * SPT, SPT)])


_sc_call = functools.partial(
    pl.kernel,
    mesh=plsc.VectorSubcoreMesh(core_axis_name="c", subcore_axis_name="s"),
    compiler_params=pltpu.CompilerParams(needs_layout_passes=False),
    out_type=(
        jax.ShapeDtypeStruct((2, ACC_ROWS, D), jnp.float32),
        jax.ShapeDtypeStruct((NW, DEG_ROWS), jnp.float32),
        jax.ShapeDtypeStruct((B, 16), jnp.int32),
        jax.ShapeDtypeStruct((S_PAD, D), jnp.float32),
    ),
    scratch_types=[
        pltpu.VMEM((N_PAD,), jnp.int32),
        pltpu.VMEM((ECHUNK,), jnp.int32),
        pltpu.VMEM((ECHUNK,), jnp.int32),
        pltpu.VMEM((ECHUNK,), jnp.int32),
        pltpu.VMEM((SEL_CAP,), jnp.int32),
        pltpu.VMEM((SEL_CAP,), jnp.int32),
        pltpu.VMEM((CHUNK, D), jnp.float32),
        pltpu.VMEM((CHUNK,), jnp.int32),
        pltpu.VMEM((DEG_ROWS,), jnp.float32),
        pltpu.VMEM((S_PAD + 32,), jnp.int32),
        pltpu.VMEM((B, 16), jnp.int32),
        pltpu.VMEM((SPT, D), jnp.float32),
        pltpu.VMEM_SHARED((ACC_ROWS, D), jnp.float32),
    ],
)(_sc_kernel)


def _tc_kernel(acc_ref, deg_ref, posl_ref, embs_ref,
               wrel_ref, wself_ref, w1_ref, b1_ref, w2_ref, b2_ref, out_ref):
    acc = acc_ref[0] + acc_ref[1]
    agg = jnp.dot(acc, wrel_ref[...], preferred_element_type=jnp.float32)
    degs = jnp.sum(deg_ref[...], axis=0)
    degb = jnp.broadcast_to(degs[0:S_PAD].reshape(1, S_PAD), (S_PAD, S_PAD))
    eye = (lax.broadcasted_iota(jnp.int32, (S_PAD, S_PAD), 0)
           == lax.broadcasted_iota(jnp.int32, (S_PAD, S_PAD), 1))
    deg = jnp.sum(jnp.where(eye, degb, 0.0), axis=1, keepdims=True)
    agg = agg / jnp.maximum(deg, 1.0)
    h = jax.nn.relu(agg + jnp.dot(embs_ref[...], wself_ref[...],
                                  preferred_element_type=jnp.float32))
    ps = posl_ref[...]
    iot = lax.broadcasted_iota(jnp.int32, (B, S_PAD), 1)
    pool = jnp.zeros((B, S_PAD), jnp.float32)
    for j in range(PPB):
        pool += (ps[:, j:j + 1] == iot).astype(jnp.float32)
    mean = jnp.dot(pool, h, preferred_element_type=jnp.float32) * (1.0 / PPB)
    hidden = jax.nn.relu(jnp.dot(mean, w1_ref[...],
                                 preferred_element_type=jnp.float32)
                         + b1_ref[...])
    out_ref[...] = jax.nn.sigmoid(
        jnp.dot(hidden, w2_ref[...], preferred_element_type=jnp.float32)
        + b2_ref[...])


def kernel(support_set, support_emb, edge_index, edge_type, node_emb,
           W_rel, W_self, W1, b1, W2, b2):
    del support_emb
    src = edge_index[0].astype(jnp.int32)
    dst = edge_index[1].astype(jnp.int32)
    typ = edge_type.astype(jnp.int32)

    idx = support_set[:, :, jnp.array([0, 2])].astype(jnp.int32).reshape(-1)
    idx_pad = jnp.concatenate(
        [idx, jnp.zeros((S_PAD + 32 - NPOS,), jnp.int32)])
    slot_map = jnp.full((N_PAD,), TRASH, jnp.int32).at[idx].set(
        jnp.arange(NPOS, dtype=jnp.int32))
    zrows = jnp.zeros((ACC_RPT, D), jnp.float32)

    acc, deg, posl, emb_slots = _sc_call(
        dst, src, typ, slot_map, node_emb, idx_pad, zrows)

    acc2 = acc.reshape(2, S_PAD, R * D)
    wrel2 = W_rel.reshape(R * D, D)

    out = pl.pallas_call(
        _tc_kernel,
        out_shape=jax.ShapeDtypeStruct((B, 1), jnp.float32),
    )(acc2, deg, posl, emb_slots, wrel2, W_self, W1,
      b1.reshape(1, D), W2, b2.reshape(1, 1))
    return out


# speedup vs baseline: 38.7246x; 1.0190x over previous
import functools

import jax
import jax.numpy as jnp
from jax import lax
from jax.experimental import pallas as pl
from jax.experimental.pallas import tpu as pltpu
from jax.experimental.pallas import tpu_sc as plsc

N = 10000
E = 320000
D = 128
R = 8
B = 64
K = 5

NPOS = B * K * 2
PPB = K * 2
TRASH = NPOS
S_PAD = 672
ACC_ROWS = S_PAD * R
N_PAD = 10240
NW = 32
EPT = E // NW
ECHUNK = 2000
CHUNK = 128
SEL_CAP = N_PAD
SPT = 24
SPT_TILES = S_PAD // SPT
ACC_RPT = ACC_ROWS // 16
DEG_ROWS = 768
DEG_RPT = DEG_ROWS // 16


def _sc_kernel(dst_h, src_h, typ_h, slot_h, emb_h, idx_h, zrows_h,
               acc_o, deg_o, posl_o, embs_o,
               slot_v, dst_v, src_v, typ_v, sela_v, sels_v,
               rows_v, rows2_v, achk_v, cnt_v, idx_v, posl_v, rows24_v,
               sem_a, sem_b, acc_sh):
    c = lax.axis_index("c")
    s = lax.axis_index("s")
    wid = s * 2 + c
    i16 = lax.iota(jnp.int32, 16)
    zero16i = i16 * 0
    trash16a = zero16i + TRASH * R

    pltpu.sync_copy(slot_h, slot_v)
    pltpu.sync_copy(idx_h, idx_v)

    pltpu.sync_copy(zrows_h, acc_sh.at[pl.ds(s * ACC_RPT, ACC_RPT)])
    zero16f = (i16 * 0).astype(jnp.float32)
    for j in range(DEG_ROWS // 16):
        cnt_v[pl.ds(j * 16, 16)] = zero16f

    plsc.subcore_barrier()

    UNROLL = 5

    def comp_chunk(ec, off):
        base = wid * EPT + ec * ECHUNK
        pltpu.sync_copy(dst_h.at[pl.ds(base, ECHUNK)], dst_v)
        pltpu.sync_copy(src_h.at[pl.ds(base, ECHUNK)], src_v)
        pltpu.sync_copy(typ_h.at[pl.ds(base, ECHUNK)], typ_v)

        def comp_body(i, off):
            for u in range(UNROLL):
                o16 = i * (16 * UNROLL) + u * 16
                d = dst_v[pl.ds(o16, 16)]
                sl = plsc.load_gather(slot_v, [d])
                r = typ_v[pl.ds(o16, 16)]
                sv = src_v[pl.ds(o16, 16)]
                a = sl * R + r
                msk = sl < TRASH
                mi = msk.astype(jnp.int32)
                pos = off + jnp.cumsum(mi) - mi
                plsc.store_scatter(sels_v, [pos], sv, mask=msk)
                plsc.store_scatter(sela_v, [pos], a, mask=msk)
                counts, lastm = plsc.scan_count(sl, mask=msk)
                plsc.addupdate_scatter(cnt_v, [sl],
                                       counts.astype(jnp.float32),
                                       mask=lastm)
                off = off + jnp.sum(mi)
            return off

        return lax.fori_loop(0, ECHUNK // (16 * UNROLL), comp_body, off)

    n_sel = lax.fori_loop(0, EPT // ECHUNK, comp_chunk, jnp.int32(0))

    for j in range(CHUNK // 16 + 1):
        pos = n_sel + j * 16 + i16
        mt = pos < SEL_CAP
        plsc.store_scatter(sels_v, [pos], zero16i, mask=mt)
        plsc.store_scatter(sela_v, [pos], trash16a, mask=mt)

    n_chunks = lax.shift_right_logical(n_sel + (CHUNK - 1), 7)

    @pl.when(n_chunks > 0)
    def _():
        pltpu.async_copy(
            emb_h.at[sels_v.at[pl.ds(0, CHUNK)]], rows_v, sem_a)

    def chunk_body(g, carry):
        for j in range(CHUNK // 16):
            achk_v[pl.ds(j * 16, 16)] = sela_v[pl.ds(g * CHUNK + j * 16, 16)]
        nxt = g + 1
        even = lax.bitwise_and(g, 1) == 0

        @pl.when(even)
        def _():
            pltpu.make_async_copy(
                emb_h.at[sels_v.at[pl.ds(g * CHUNK, CHUNK)]], rows_v,
                sem_a).wait()

            @pl.when(nxt < n_chunks)
            def _():
                pltpu.async_copy(
                    emb_h.at[sels_v.at[pl.ds(nxt * CHUNK, CHUNK)]], rows2_v,
                    sem_b)

            pltpu.sync_copy(rows_v, acc_sh.at[achk_v], add=True)

        @pl.when(jnp.logical_not(even))
        def _():
            pltpu.make_async_copy(
                emb_h.at[sels_v.at[pl.ds(g * CHUNK, CHUNK)]], rows2_v,
                sem_b).wait()

            @pl.when(nxt < n_chunks)
            def _():
                pltpu.async_copy(
                    emb_h.at[sels_v.at[pl.ds(nxt * CHUNK, CHUNK)]], rows_v,
                    sem_a)

            pltpu.sync_copy(rows2_v, acc_sh.at[achk_v], add=True)

        return carry

    lax.fori_loop(0, n_chunks, chunk_body, jnp.int32(0))

    @pl.when(wid == 0)
    def _():
        for j in range(B):
            posl_v[j, pl.ds(0, 16)] = zero16i + TRASH
        for g in range(NPOS // 16):
            sl = plsc.load_gather(slot_v, [idx_v[pl.ds(g * 16, 16)]])
            p = i16 + g * 16
            rvec = lax.shift_right_logical(p * 6554, 16)
            cvec = p - rvec * PPB
            plsc.store_scatter(posl_v, [rvec, cvec], sl)
        pltpu.sync_copy(posl_v, posl_o)

    @pl.when(wid < SPT_TILES)
    def _():
        pltpu.sync_copy(emb_h.at[idx_v.at[pl.ds(wid * SPT, SPT)]], rows24_v)

    plsc.subcore_barrier()

    pltpu.sync_copy(acc_sh.at[pl.ds(s * ACC_RPT, ACC_RPT)],
                    acc_o.at[c, pl.ds(s * ACC_RPT, ACC_RPT)])
    pltpu.sync_copy(cnt_v, deg_o.at[wid])

    @pl.when(wid < SPT_TILES)
    def _():
        pltpu.sync_copy(rows24_v, embs_o.at[pl.ds(wid * SPT, SPT)])


_sc_call = functools.partial(
    pl.kernel,
    mesh=plsc.VectorSubcoreMesh(core_axis_name="c", subcore_axis_name="s"),
    compiler_params=pltpu.CompilerParams(needs_layout_passes=False),
    out_type=(
        jax.ShapeDtypeStruct((2, ACC_ROWS, D), jnp.float32),
        jax.ShapeDtypeStruct((NW, DEG_ROWS), jnp.float32),
        jax.ShapeDtypeStruct((B, 16), jnp.int32),
        jax.ShapeDtypeStruct((S_PAD, D), jnp.float32),
    ),
    scratch_types=[
        pltpu.VMEM((N_PAD,), jnp.int32),
        pltpu.VMEM((ECHUNK,), jnp.int32),
        pltpu.VMEM((ECHUNK,), jnp.int32),
        pltpu.VMEM((ECHUNK,), jnp.int32),
        pltpu.VMEM((SEL_CAP,), jnp.int32),
        pltpu.VMEM((SEL_CAP,), jnp.int32),
        pltpu.VMEM((CHUNK, D), jnp.float32),
        pltpu.VMEM((CHUNK, D), jnp.float32),
        pltpu.VMEM((CHUNK,), jnp.int32),
        pltpu.VMEM((DEG_ROWS,), jnp.float32),
        pltpu.VMEM((S_PAD + 32,), jnp.int32),
        pltpu.VMEM((B, 16), jnp.int32),
        pltpu.VMEM((SPT, D), jnp.float32),
        pltpu.SemaphoreType.DMA,
        pltpu.SemaphoreType.DMA,
        pltpu.VMEM_SHARED((ACC_ROWS, D), jnp.float32),
    ],
)(_sc_kernel)


def _tc_kernel(acc_ref, deg_ref, posl_ref, embs_ref,
               wrel_ref, wself_ref, w1_ref, b1_ref, w2_ref, b2_ref, out_ref):
    acc = acc_ref[0] + acc_ref[1]
    agg = jnp.dot(acc, wrel_ref[...], preferred_element_type=jnp.float32)
    degs = jnp.sum(deg_ref[...], axis=0)
    degb = jnp.broadcast_to(degs[0:S_PAD].reshape(1, S_PAD), (S_PAD, S_PAD))
    eye = (lax.broadcasted_iota(jnp.int32, (S_PAD, S_PAD), 0)
           == lax.broadcasted_iota(jnp.int32, (S_PAD, S_PAD), 1))
    deg = jnp.sum(jnp.where(eye, degb, 0.0), axis=1, keepdims=True)
    agg = agg / jnp.maximum(deg, 1.0)
    h = jax.nn.relu(agg + jnp.dot(embs_ref[...], wself_ref[...],
                                  preferred_element_type=jnp.float32))
    ps = posl_ref[...]
    iot = lax.broadcasted_iota(jnp.int32, (B, S_PAD), 1)
    pool = jnp.zeros((B, S_PAD), jnp.float32)
    for j in range(PPB):
        pool += (ps[:, j:j + 1] == iot).astype(jnp.float32)
    mean = jnp.dot(pool, h, preferred_element_type=jnp.float32) * (1.0 / PPB)
    hidden = jax.nn.relu(jnp.dot(mean, w1_ref[...],
                                 preferred_element_type=jnp.float32)
                         + b1_ref[...])
    out_ref[...] = jax.nn.sigmoid(
        jnp.dot(hidden, w2_ref[...], preferred_element_type=jnp.float32)
        + b2_ref[...])


def kernel(support_set, support_emb, edge_index, edge_type, node_emb,
           W_rel, W_self, W1, b1, W2, b2):
    del support_emb
    src = edge_index[0].astype(jnp.int32)
    dst = edge_index[1].astype(jnp.int32)
    typ = edge_type.astype(jnp.int32)

    idx = support_set[:, :, jnp.array([0, 2])].astype(jnp.int32).reshape(-1)
    idx_pad = jnp.concatenate(
        [idx, jnp.zeros((S_PAD + 32 - NPOS,), jnp.int32)])
    slot_map = jnp.full((N_PAD,), TRASH, jnp.int32).at[idx].set(
        jnp.arange(NPOS, dtype=jnp.int32))
    zrows = jnp.zeros((ACC_RPT, D), jnp.float32)

    acc, deg, posl, emb_slots = _sc_call(
        dst, src, typ, slot_map, node_emb, idx_pad, zrows)

    acc2 = acc.reshape(2, S_PAD, R * D)
    wrel2 = W_rel.reshape(R * D, D)

    out = pl.pallas_call(
        _tc_kernel,
        out_shape=jax.ShapeDtypeStruct((B, 1), jnp.float32),
    )(acc2, deg, posl, emb_slots, wrel2, W_self, W1,
      b1.reshape(1, D), W2, b2.reshape(1, 1))
    return out
